# Initial kernel scaffold; baseline (speedup 1.0000x reference)
import jax
import jax.numpy as jnp
from jax.experimental import pallas as pl
from jax.experimental.pallas import tpu as pltpu

N = 100000
C = 100
D = 128
BLK = 5000


def _seg_body(logits_ref, feats_ref, sums_ref, counts_ref):
    i = pl.program_id(0)
    x = logits_ref[...]
    lbl = jnp.argmax(x, axis=1)
    iota = jax.lax.broadcasted_iota(jnp.int32, (1, C), 1)
    onehot = (lbl[:, None] == iota).astype(jnp.float32)
    psums = jax.lax.dot_general(
        onehot, feats_ref[...], (((0,), (0,)), ((), ())),
        preferred_element_type=jnp.float32)
    pcounts = jnp.sum(onehot, axis=0)[None, :]

    @pl.when(i == 0)
    def _():
        sums_ref[...] = psums
        counts_ref[...] = pcounts

    @pl.when(i > 0)
    def _():
        sums_ref[...] += psums
        counts_ref[...] += pcounts


def _loss_body(sums_ref, counts_ref, out_ref):
    cnt = counts_ref[0, :]
    recip = 1.0 / jnp.maximum(cnt, 1.0)
    means = sums_ref[...] * recip[:, None]
    sq = jnp.sum(means * means, axis=1, keepdims=True)
    norm = jnp.maximum(jnp.sqrt(sq), 1e-12)
    normed = means / norm
    cos = jax.lax.dot_general(
        normed, normed, (((1,), (1,)), ((), ())),
        preferred_element_type=jnp.float32)
    present = (cnt > 0.0).astype(jnp.float32)
    pm = present[:, None] * present[None, :]
    ri = jax.lax.broadcasted_iota(jnp.int32, (C, C), 0)
    ci = jax.lax.broadcasted_iota(jnp.int32, (C, C), 1)
    offdiag = (ri != ci).astype(jnp.float32)
    loss = (1.0 - cos) * pm * offdiag
    out_ref[0, 0] = jnp.sum(loss)


def kernel(logits, img_feats):
    sums, counts = pl.pallas_call(
        _seg_body,
        grid=(N // BLK,),
        in_specs=[
            pl.BlockSpec((BLK, C), lambda i: (i, 0)),
            pl.BlockSpec((BLK, D), lambda i: (i, 0)),
        ],
        out_specs=[
            pl.BlockSpec((C, D), lambda i: (0, 0)),
            pl.BlockSpec((1, C), lambda i: (0, 0)),
        ],
        out_shape=[
            jax.ShapeDtypeStruct((C, D), jnp.float32),
            jax.ShapeDtypeStruct((1, C), jnp.float32),
        ],
        compiler_params=pltpu.CompilerParams(
            dimension_semantics=("arbitrary",)),
    )(logits, img_feats)

    out = pl.pallas_call(
        _loss_body,
        out_shape=jax.ShapeDtypeStruct((1, 1), jnp.float32),
    )(sums, counts)
    return out[0, 0]


# speedup vs baseline: 5.2050x; 5.2050x over previous
import jax
import jax.numpy as jnp
from jax.experimental import pallas as pl
from jax.experimental.pallas import tpu as pltpu

N = 100000
C = 100
D = 128
BLK = 5000


def _seg_body(logits_ref, feats_ref, sums_ref, counts_ref):
    i = pl.program_id(0)
    x = logits_ref[...]
    lbl = jnp.argmax(x, axis=1)
    iota = jax.lax.broadcasted_iota(jnp.int32, (1, C), 1)
    onehot = (lbl[:, None] == iota).astype(jnp.float32)
    psums = jax.lax.dot_general(
        onehot, feats_ref[...], (((0,), (0,)), ((), ())),
        preferred_element_type=jnp.float32)
    pcounts = jnp.sum(onehot, axis=0)[None, :]

    @pl.when(i == 0)
    def _():
        sums_ref[...] = psums
        counts_ref[...] = pcounts

    @pl.when(i > 0)
    def _():
        sums_ref[...] += psums
        counts_ref[...] += pcounts


def _loss_body(sums_ref, counts_ref, out_ref):
    cnt = counts_ref[0, :]
    recip = 1.0 / jnp.maximum(cnt, 1.0)
    means = sums_ref[...] * recip[:, None]
    sq = jnp.sum(means * means, axis=1, keepdims=True)
    norm = jnp.maximum(jnp.sqrt(sq), 1e-12)
    normed = means / norm
    cos = jax.lax.dot_general(
        normed, normed, (((1,), (1,)), ((), ())),
        preferred_element_type=jnp.float32)
    present = (cnt > 0.0).astype(jnp.float32)
    pm = present[:, None] * present[None, :]
    ri = jax.lax.broadcasted_iota(jnp.int32, (C, C), 0)
    ci = jax.lax.broadcasted_iota(jnp.int32, (C, C), 1)
    offdiag = (ri != ci).astype(jnp.float32)
    loss = (1.0 - cos) * pm * offdiag
    out_ref[...] = jnp.sum(loss).reshape(1, 1)


def kernel(logits, img_feats):
    sums, counts = pl.pallas_call(
        _seg_body,
        grid=(N // BLK,),
        in_specs=[
            pl.BlockSpec((BLK, C), lambda i: (i, 0)),
            pl.BlockSpec((BLK, D), lambda i: (i, 0)),
        ],
        out_specs=[
            pl.BlockSpec((C, D), lambda i: (0, 0)),
            pl.BlockSpec((1, C), lambda i: (0, 0)),
        ],
        out_shape=[
            jax.ShapeDtypeStruct((C, D), jnp.float32),
            jax.ShapeDtypeStruct((1, C), jnp.float32),
        ],
        compiler_params=pltpu.CompilerParams(
            dimension_semantics=("arbitrary",)),
    )(logits, img_feats)

    out = pl.pallas_call(
        _loss_body,
        out_shape=jax.ShapeDtypeStruct((1, 1), jnp.float32),
    )(sums, counts)
    return out[0, 0]
